# Initial kernel scaffold; baseline (speedup 1.0000x reference)
import functools

import jax
import jax.numpy as jnp
from jax import lax
from jax.experimental import pallas as pl
from jax.experimental.pallas import tpu as pltpu
from jax.experimental.pallas import tpu_sc as plsc

_POOL = 1000
_POOL_PAD = 1024
_EMBED = 128
_LENGTH = 8
_TOPK = 4
_BATCH = 1024

_BR = 128
_GRID = _BATCH // _BR

_NC = 2
_NS = 16
_NW = _NC * _NS
_BPW = (_BATCH * _TOPK) // _NW
_NCHUNK = 4
_CH = _BPW // _NCHUNK
_ROW = _LENGTH * _EMBED


def _topk_body(q_ref, keys_ref, idx_ref, loss_ref):
    q = q_ref[...]
    keys = keys_ref[...]
    q2 = jnp.sum(q * q, axis=1, keepdims=True)
    k2 = jnp.sum(keys * keys, axis=1)[None, :]
    qk = lax.dot_general(q, keys, (((1,), (1,)), ((), ())),
                         preferred_element_type=jnp.float32)
    d2 = jnp.maximum(q2 + k2 - 2.0 * qk, 0.0)
    d = jnp.sqrt(d2)
    col = lax.broadcasted_iota(jnp.int32, d.shape, 1)
    big = jnp.float32(1e30)
    d = jnp.where(col < _POOL, d, big)

    total = jnp.float32(0.0)
    picks = []
    for _ in range(_TOPK):
        m = jnp.min(d, axis=1, keepdims=True)
        am = jnp.min(jnp.where(d == m, col, jnp.int32(2**30)),
                     axis=1, keepdims=True)
        picks.append(am)
        total = total + jnp.sum(m)
        d = jnp.where(col == am, big, d)
    idx_ref[...] = jnp.concatenate(picks, axis=1)
    loss_ref[0, 0] = total


def _topk_call(query, keys_pad):
    return pl.pallas_call(
        _topk_body,
        grid=(_GRID,),
        in_specs=[
            pl.BlockSpec((_BR, _EMBED), lambda i: (i, 0)),
            pl.BlockSpec((_POOL_PAD, _EMBED), lambda i: (0, 0)),
        ],
        out_specs=[
            pl.BlockSpec((_BR, _TOPK), lambda i: (i, 0)),
            pl.BlockSpec((1, 1), lambda i: (i, 0)),
        ],
        out_shape=[
            jax.ShapeDtypeStruct((_BATCH, _TOPK), jnp.int32),
            jax.ShapeDtypeStruct((_GRID, 1), jnp.float32),
        ],
    )(query, keys_pad)


def _gather_body(table_hbm, idx_hbm, out_hbm, idx_v, rows_v, sem0, sem1):
    wid = lax.axis_index("s") * _NC + lax.axis_index("c")
    base = wid * _BPW
    pltpu.sync_copy(idx_hbm.at[wid], idx_v)
    sems = (sem0, sem1)
    cps = [None, None]
    for c in range(_NCHUNK):
        b = c % 2
        if cps[b] is not None:
            cps[b].wait()
            pltpu.sync_copy(rows_v.at[b],
                            out_hbm.at[pl.ds(base + (c - 2) * _CH, _CH)])
        cps[b] = pltpu.async_copy(table_hbm.at[idx_v.at[c]],
                                  rows_v.at[b], sems[b])
    for c in range(_NCHUNK - 2, _NCHUNK):
        b = c % 2
        cps[b].wait()
        pltpu.sync_copy(rows_v.at[b],
                        out_hbm.at[pl.ds(base + c * _CH, _CH)])


def _gather_call(table, idx3):
    mesh = plsc.VectorSubcoreMesh(core_axis_name="c", subcore_axis_name="s")
    return pl.kernel(
        _gather_body,
        out_type=jax.ShapeDtypeStruct((_BATCH * _TOPK, _ROW), jnp.float32),
        mesh=mesh,
        scratch_types=[
            pltpu.VMEM((_NCHUNK, _CH), jnp.int32),
            pltpu.VMEM((2, _CH, _ROW), jnp.float32),
            pltpu.SemaphoreType.DMA,
            pltpu.SemaphoreType.DMA,
        ],
    )(table, idx3)


@jax.jit
def kernel(query, prompt_keys, prompt_values):
    keys_pad = jnp.pad(prompt_keys, ((0, _POOL_PAD - _POOL), (0, 0)))
    idx, loss_parts = _topk_call(query, keys_pad)
    key_loss = jnp.sum(loss_parts) / _BATCH
    table = prompt_values.reshape(_POOL, _ROW)
    idx3 = idx.reshape(_NW, _NCHUNK, _CH)
    rows = _gather_call(table, idx3)
    quantized = rows.reshape(_BATCH, _TOPK, _LENGTH, _EMBED)
    return (quantized, key_loss)


# speedup vs baseline: 1.0959x; 1.0959x over previous
import functools

import jax
import jax.numpy as jnp
from jax import lax
from jax.experimental import pallas as pl
from jax.experimental.pallas import tpu as pltpu
from jax.experimental.pallas import tpu_sc as plsc

_POOL = 1000
_POOL_PAD = 1024
_EMBED = 128
_LENGTH = 8
_TOPK = 4
_BATCH = 1024

_BR = 128
_GRID = _BATCH // _BR

_NC = 2
_NS = 16
_NW = _NC * _NS
_BPW = (_BATCH * _TOPK) // _NW
_NCHUNK = 4
_CH = _BPW // _NCHUNK
_ROW = _LENGTH * _EMBED


def _topk_body(q_ref, keys_ref, idx_ref, loss_ref):
    q = q_ref[...]
    keys = keys_ref[...]
    q2 = jnp.sum(q * q, axis=1, keepdims=True)
    k2 = jnp.sum(keys * keys, axis=1)[None, :]
    qk = lax.dot_general(q, keys, (((1,), (1,)), ((), ())),
                         preferred_element_type=jnp.float32)
    d2 = jnp.maximum(q2 + k2 - 2.0 * qk, 0.0)
    d = jnp.sqrt(d2)
    col = lax.broadcasted_iota(jnp.int32, d.shape, 1)
    big = jnp.float32(1e30)
    d = jnp.where(col < _POOL, d, big)

    total = jnp.float32(0.0)
    picks = []
    for _ in range(_TOPK):
        m = jnp.min(d, axis=1, keepdims=True)
        am = jnp.min(jnp.where(d == m, col, jnp.int32(2**30)),
                     axis=1, keepdims=True)
        picks.append(am)
        total = total + jnp.sum(m)
        d = jnp.where(col == am, big, d)
    idx_ref[...] = jnp.concatenate(picks, axis=1)
    loss_ref[...] = jnp.broadcast_to(total, (1, 1, 128))


def _topk_call(query, keys_pad):
    return pl.pallas_call(
        _topk_body,
        grid=(_GRID,),
        in_specs=[
            pl.BlockSpec((_BR, _EMBED), lambda i: (i, 0)),
            pl.BlockSpec((_POOL_PAD, _EMBED), lambda i: (0, 0)),
        ],
        out_specs=[
            pl.BlockSpec((_BR, _TOPK), lambda i: (i, 0)),
            pl.BlockSpec((1, 1, 128), lambda i: (i, 0, 0)),
        ],
        out_shape=[
            jax.ShapeDtypeStruct((_BATCH, _TOPK), jnp.int32),
            jax.ShapeDtypeStruct((_GRID, 1, 128), jnp.float32),
        ],
    )(query, keys_pad)


def _gather_body(table_hbm, idx_hbm, out_hbm, idx_v, rows_v, sem0, sem1):
    wid = lax.axis_index("s") * _NC + lax.axis_index("c")
    base = wid * _BPW
    pltpu.sync_copy(idx_hbm.at[wid], idx_v)
    sems = (sem0, sem1)
    cps = [None, None]
    for c in range(_NCHUNK):
        b = c % 2
        if cps[b] is not None:
            cps[b].wait()
            pltpu.sync_copy(rows_v.at[b],
                            out_hbm.at[pl.ds(base + (c - 2) * _CH, _CH)])
        cps[b] = pltpu.async_copy(table_hbm.at[idx_v.at[c]],
                                  rows_v.at[b], sems[b])
    for c in range(_NCHUNK - 2, _NCHUNK):
        b = c % 2
        cps[b].wait()
        pltpu.sync_copy(rows_v.at[b],
                        out_hbm.at[pl.ds(base + c * _CH, _CH)])


def _gather_call(table, idx3):
    mesh = plsc.VectorSubcoreMesh(core_axis_name="c", subcore_axis_name="s")
    return pl.kernel(
        _gather_body,
        out_type=jax.ShapeDtypeStruct((_BATCH * _TOPK, _ROW), jnp.float32),
        mesh=mesh,
        scratch_types=[
            pltpu.VMEM((_NCHUNK, _CH), jnp.int32),
            pltpu.VMEM((2, _CH, _ROW), jnp.float32),
            pltpu.SemaphoreType.DMA,
            pltpu.SemaphoreType.DMA,
        ],
    )(table, idx3)


@jax.jit
def kernel(query, prompt_keys, prompt_values):
    keys_pad = jnp.pad(prompt_keys, ((0, _POOL_PAD - _POOL), (0, 0)))
    idx, loss_parts = _topk_call(query, keys_pad)
    key_loss = jnp.sum(loss_parts[:, 0, 0]) / _BATCH
    table = prompt_values.reshape(_POOL, _ROW)
    idx3 = idx.reshape(_NW, _NCHUNK, _CH)
    rows = _gather_call(table, idx3)
    quantized = rows.reshape(_BATCH, _TOPK, _LENGTH, _EMBED)
    return (quantized, key_loss)


# speedup vs baseline: 1.7343x; 1.5825x over previous
import functools

import jax
import jax.numpy as jnp
from jax import lax
from jax.experimental import pallas as pl
from jax.experimental.pallas import tpu as pltpu
from jax.experimental.pallas import tpu_sc as plsc

_POOL = 1000
_POOL_PAD = 1024
_EMBED = 128
_LENGTH = 8
_TOPK = 4
_BATCH = 1024

_BR = 256
_GRID = _BATCH // _BR

_NC = 2
_NS = 16
_NW = _NC * _NS
_BPW = (_BATCH * _TOPK) // _NW
_NCHUNK = 4
_CH = _BPW // _NCHUNK


def _topk_body(q_ref, keys_ref, idx_ref, loss_ref):
    q = q_ref[...]
    keys = keys_ref[...]
    q2 = jnp.sum(q * q, axis=1, keepdims=True)
    k2 = jnp.sum(keys * keys, axis=1)[None, :]
    qk = lax.dot_general(q, keys, (((1,), (1,)), ((), ())),
                         preferred_element_type=jnp.float32)
    d2 = jnp.maximum(q2 + k2 - 2.0 * qk, 0.0)
    col = lax.broadcasted_iota(jnp.int32, d2.shape, 1)
    big = jnp.float32(1e30)
    d2 = jnp.where(col < _POOL, d2, big)

    total = jnp.float32(0.0)
    picks = []
    for _ in range(_TOPK):
        m = jnp.min(d2, axis=1, keepdims=True)
        am = jnp.min(jnp.where(d2 == m, col, jnp.int32(2**30)),
                     axis=1, keepdims=True)
        picks.append(am)
        total = total + jnp.sum(jnp.sqrt(m))
        d2 = jnp.where(col == am, big, d2)
    idx_ref[...] = jnp.concatenate(picks, axis=1)
    loss_ref[...] = jnp.broadcast_to(total, (1, 1, 128))


def _topk_call(query, keys_pad):
    return pl.pallas_call(
        _topk_body,
        grid=(_GRID,),
        in_specs=[
            pl.BlockSpec((_BR, _EMBED), lambda i: (i, 0)),
            pl.BlockSpec((_POOL_PAD, _EMBED), lambda i: (0, 0)),
        ],
        out_specs=[
            pl.BlockSpec((_BR, _TOPK), lambda i: (i, 0)),
            pl.BlockSpec((1, 1, 128), lambda i: (i, 0, 0)),
        ],
        out_shape=[
            jax.ShapeDtypeStruct((_BATCH, _TOPK), jnp.int32),
            jax.ShapeDtypeStruct((_GRID, 1, 128), jnp.float32),
        ],
    )(query, keys_pad)


def _gather_body(table_hbm, idx_hbm, out_hbm, idx_v, rows_v, sem0, sem1):
    wid = lax.axis_index("s") * _NC + lax.axis_index("c")
    base = wid * _BPW
    pltpu.sync_copy(idx_hbm.at[wid], idx_v)
    sems = (sem0, sem1)
    cps = [None, None]
    for c in range(_NCHUNK):
        b = c % 2
        if cps[b] is not None:
            cps[b].wait()
            pltpu.sync_copy(rows_v.at[b],
                            out_hbm.at[pl.ds(base + (c - 2) * _CH, _CH)])
        cps[b] = pltpu.async_copy(
            table_hbm.at[idx_v.at[pl.ds(c * _CH, _CH)]],
            rows_v.at[b], sems[b])
    for c in range(_NCHUNK - 2, _NCHUNK):
        b = c % 2
        cps[b].wait()
        pltpu.sync_copy(rows_v.at[b],
                        out_hbm.at[pl.ds(base + c * _CH, _CH)])


def _gather_call(table, idx2):
    mesh = plsc.VectorSubcoreMesh(core_axis_name="c", subcore_axis_name="s")
    return pl.kernel(
        _gather_body,
        out_type=jax.ShapeDtypeStruct((_BATCH * _TOPK, _LENGTH, _EMBED),
                                      jnp.float32),
        mesh=mesh,
        scratch_types=[
            pltpu.VMEM((_BPW,), jnp.int32),
            pltpu.VMEM((2, _CH, _LENGTH, _EMBED), jnp.float32),
            pltpu.SemaphoreType.DMA,
            pltpu.SemaphoreType.DMA,
        ],
    )(table, idx2)


@jax.jit
def kernel(query, prompt_keys, prompt_values):
    keys_pad = jnp.pad(prompt_keys, ((0, _POOL_PAD - _POOL), (0, 0)))
    idx, loss_parts = _topk_call(query, keys_pad)
    idx2 = idx.reshape(_NW, _BPW)
    key_loss = jnp.sum(loss_parts[:, 0, 0]) / _BATCH
    rows = _gather_call(prompt_values, idx2)
    quantized = rows.reshape(_BATCH, _TOPK, _LENGTH, _EMBED)
    return (quantized, key_loss)
